# Initial kernel scaffold; baseline (speedup 1.0000x reference)
import jax
import jax.numpy as jnp
from jax.experimental import pallas as pl


def kernel(x, edge_index, edge_attr, W0, b0, W1, b1, mix_W, mix_b, gru_Wih, gru_Whh, gru_bih, gru_bhh, ro_W, ro_b):
    raise NotImplementedError("write your pallas kernel here")


# speedup vs baseline: 1.4666x; 1.4666x over previous
import functools

import jax
import jax.numpy as jnp
from jax import lax
from jax.experimental import pallas as pl
from jax.experimental.pallas import tpu as pltpu

_N = 10000
_E = 320000
_HID = 256
_ROWS_BLK = 2000


def _dense_body(s_ref, W2_ref, b2_ref, mixW_ref, mixb_ref, Wih_ref, bih_ref,
                bhh_ref, roW_ref, rob_ref, h_ref, o_ref):
    s = s_ref[...]
    u = jnp.dot(s, W2_ref[...], preferred_element_type=jnp.float32) + b2_ref[...]
    h = jnp.maximum(
        jnp.dot(u, mixW_ref[...], preferred_element_type=jnp.float32) + mixb_ref[...],
        0.0)
    gi = jnp.dot(h, Wih_ref[...], preferred_element_type=jnp.float32) + bih_ref[...]
    bhh = bhh_ref[...]
    r = jax.nn.sigmoid(gi[:, 0:_HID] + bhh[:, 0:_HID])
    z = jax.nn.sigmoid(gi[:, _HID:2 * _HID] + bhh[:, _HID:2 * _HID])
    n = jnp.tanh(gi[:, 2 * _HID:] + r * bhh[:, 2 * _HID:])
    hn = (1.0 - z) * n
    h_ref[...] = hn
    o_ref[...] = jnp.dot(hn, roW_ref[...], preferred_element_type=jnp.float32) + rob_ref[...]


def _dense_stage(s_cat, W2, b2, mix_W, mix_b, Wih, bih, bhh, ro_W, ro_b):
    grid = (_N // _ROWS_BLK,)
    full = lambda shape: pl.BlockSpec(shape, lambda i: (0, 0))
    return pl.pallas_call(
        _dense_body,
        grid=grid,
        in_specs=[
            pl.BlockSpec((_ROWS_BLK, 256), lambda i: (i, 0)),
            full((256, 512)),
            full((1, 512)),
            full((512, 256)),
            full((1, 256)),
            full((256, 768)),
            full((1, 768)),
            full((1, 768)),
            full((256, 256)),
            full((1, 256)),
        ],
        out_specs=[
            pl.BlockSpec((_ROWS_BLK, 256), lambda i: (i, 0)),
            pl.BlockSpec((_ROWS_BLK, 256), lambda i: (i, 0)),
        ],
        out_shape=[
            jax.ShapeDtypeStruct((_N, 256), jnp.float32),
            jax.ShapeDtypeStruct((_N, 256), jnp.float32),
        ],
    )(s_cat, W2, b2, mix_W, mix_b, Wih, bih, bhh, ro_W, ro_b)


def kernel(x, edge_index, edge_attr, W0, b0, W1, b1, mix_W, mix_b,
           gru_Wih, gru_Whh, gru_bih, gru_bhh, ro_W, ro_b):
    del gru_Whh
    src = edge_index[0]
    dst = edge_index[1]

    ss = []
    for ety in range(2):
        w = (edge_attr == ety).astype(jnp.float32)
        D = jax.ops.segment_sum(w, src, num_segments=_N)
        Dinv = jnp.where(D > 0, 1.0 / D, 0.0)
        B = jax.ops.segment_sum(w, dst, num_segments=_N)
        Binv = jnp.where(B > 0, 1.0 / B, 0.0)
        e_pre = jax.ops.segment_sum(w[:, None] * x[src], dst, num_segments=_N)
        t = Binv[:, None] * e_pre
        n_pre = jax.ops.segment_sum(w[:, None] * t[dst], src, num_segments=_N)
        ss.append(Dinv[:, None] * n_pre)
    s_cat = jnp.concatenate(ss, axis=1)

    W2 = jnp.zeros((256, 512), jnp.float32)
    W2 = W2.at[:128, :256].set(W0).at[128:, 256:].set(W1)
    b2 = jnp.concatenate([b0, b1])[None, :]

    h_next, o = _dense_stage(
        s_cat, W2, b2, mix_W, mix_b[None, :], gru_Wih, gru_bih[None, :],
        gru_bhh[None, :], ro_W, ro_b[None, :])
    return (h_next, o[:, :3])


# speedup vs baseline: 5.9759x; 4.0746x over previous
import functools

import jax
import jax.numpy as jnp
from jax import lax
from jax.experimental import pallas as pl
from jax.experimental.pallas import tpu as pltpu
from jax.experimental.pallas import tpu_sc as plsc

_N = 10000
_E = 320000
_HID = 256
_ROWS_BLK = 2000

_NSC = 2
_NT = 16
_NR = 20480
_RT = _NR // _NT
_EP = _E // _NT
_K = 80
_SB = 80


def _sc_pass_body(table, gidx2, cidx, out, out_hist, acc, hist, gbuf, dbuf,
                  rows, sb, histv, onesv, sem):
    c = lax.axis_index("c")
    s = lax.axis_index("s")
    r0 = s * _RT
    e0 = s * _EP
    z16 = jnp.zeros((16,), jnp.float32)
    one16 = jnp.ones((16,), jnp.float32)
    n_strips = _RT // _SB

    def _zrow(i, carry):
        for j in range(4):
            sb[i, pl.ds(j * 16, 16)] = z16
        return carry
    lax.fori_loop(0, _SB, _zrow, 0)

    def _zhist(i, carry):
        histv[pl.ds(i * 16, 16)] = z16
        return carry
    lax.fori_loop(0, _RT // 16, _zhist, 0)
    for j in range(_K // 16):
        onesv[pl.ds(j * 16, 16)] = one16

    def _zstrip(st, carry):
        pltpu.sync_copy(sb, acc.at[pl.ds(r0 + st * _SB, _SB)])
        return carry
    lax.fori_loop(0, n_strips, _zstrip, 0)
    pltpu.sync_copy(histv, hist.at[pl.ds(r0, _RT)])
    plsc.subcore_barrier()

    def _chunk(j, carry):
        off = e0 + j * _K
        pltpu.sync_copy(gidx2.at[pl.ds(c * _E + off, _K)], gbuf)
        pltpu.sync_copy(cidx.at[pl.ds(off, _K)], dbuf)
        pltpu.async_copy(table.at[gbuf], rows, sem).wait()
        pltpu.sync_copy(rows, acc.at[dbuf], add=True)
        pltpu.sync_copy(onesv, hist.at[dbuf], add=True)
        return carry
    lax.fori_loop(0, _EP // _K, _chunk, 0)

    plsc.subcore_barrier()

    def _fstrip(st, carry):
        pltpu.sync_copy(acc.at[pl.ds(r0 + st * _SB, _SB)], sb)
        pltpu.sync_copy(sb, out.at[pl.ds(c * _NR + r0 + st * _SB, _SB)])
        return carry
    lax.fori_loop(0, n_strips, _fstrip, 0)

    @pl.when(c == 0)
    def _():
        pltpu.sync_copy(hist.at[pl.ds(r0, _RT)], histv)
        pltpu.sync_copy(histv, out_hist.at[pl.ds(r0, _RT)])


def _sc_pass(table, gidx2, cidx):
    mesh = plsc.VectorSubcoreMesh(core_axis_name="c", subcore_axis_name="s")
    f = pl.kernel(
        _sc_pass_body,
        mesh=mesh,
        out_type=[
            jax.ShapeDtypeStruct((_NSC * _NR, 64), jnp.float32),
            jax.ShapeDtypeStruct((_NR,), jnp.float32),
        ],
        scratch_types=[
            pltpu.VMEM_SHARED((_NR, 64), jnp.float32),
            pltpu.VMEM_SHARED((_NR,), jnp.float32),
            pltpu.VMEM((_K,), jnp.int32),
            pltpu.VMEM((_K,), jnp.int32),
            pltpu.VMEM((_K, 64), jnp.float32),
            pltpu.VMEM((_SB, 64), jnp.float32),
            pltpu.VMEM((_RT,), jnp.float32),
            pltpu.VMEM((_K,), jnp.float32),
            pltpu.SemaphoreType.DMA,
        ],
        compiler_params=pltpu.CompilerParams(use_tc_tiling_on_sc=False),
    )
    return f(table, gidx2, cidx)


def _scale_body(a_ref, h_ref, o_ref):
    h = h_ref[...]
    o_ref[...] = a_ref[...] * jnp.where(h > 0.0, 1.0 / h, 0.0)


def _scale_stage(acc, histcat):
    blk = 4096
    grid = (_NSC * _NR // blk,)
    return pl.pallas_call(
        _scale_body,
        grid=grid,
        in_specs=[
            pl.BlockSpec((blk, 64), lambda i: (i, 0)),
            pl.BlockSpec((blk, 1), lambda i: (i, 0)),
        ],
        out_specs=pl.BlockSpec((blk, 64), lambda i: (i, 0)),
        out_shape=jax.ShapeDtypeStruct((_NSC * _NR, 64), jnp.float32),
    )(acc, histcat)


def _dense_body(s_ref, d0_ref, d1_ref, W2_ref, b2_ref, mixW_ref, mixb_ref,
                Wih_ref, bih_ref, bhh_ref, roW_ref, rob_ref, h_ref, o_ref):
    d0 = d0_ref[...]
    d1 = d1_ref[...]
    inv0 = jnp.where(d0 > 0.0, 1.0 / d0, 0.0)
    inv1 = jnp.where(d1 > 0.0, 1.0 / d1, 0.0)
    s0 = s_ref[...]
    s = jnp.concatenate([s0[:, :128] * inv0, s0[:, 128:] * inv1], axis=1)
    u = jnp.dot(s, W2_ref[...], preferred_element_type=jnp.float32) + b2_ref[...]
    h = jnp.maximum(
        jnp.dot(u, mixW_ref[...], preferred_element_type=jnp.float32) + mixb_ref[...],
        0.0)
    gi = jnp.dot(h, Wih_ref[...], preferred_element_type=jnp.float32) + bih_ref[...]
    bhh = bhh_ref[...]
    r = jax.nn.sigmoid(gi[:, 0:_HID] + bhh[:, 0:_HID])
    z = jax.nn.sigmoid(gi[:, _HID:2 * _HID] + bhh[:, _HID:2 * _HID])
    n = jnp.tanh(gi[:, 2 * _HID:] + r * bhh[:, 2 * _HID:])
    hn = (1.0 - z) * n
    h_ref[...] = hn
    o_ref[...] = jnp.dot(hn, roW_ref[...], preferred_element_type=jnp.float32) + rob_ref[...]


def _dense_stage(s_cat, d0, d1, W2, b2, mix_W, mix_b, Wih, bih, bhh, ro_W, ro_b):
    grid = (_N // _ROWS_BLK,)
    full = lambda shape: pl.BlockSpec(shape, lambda i: (0, 0))
    return pl.pallas_call(
        _dense_body,
        grid=grid,
        in_specs=[
            pl.BlockSpec((_ROWS_BLK, 256), lambda i: (i, 0)),
            pl.BlockSpec((_ROWS_BLK, 1), lambda i: (i, 0)),
            pl.BlockSpec((_ROWS_BLK, 1), lambda i: (i, 0)),
            full((256, 512)),
            full((1, 512)),
            full((512, 256)),
            full((1, 256)),
            full((256, 768)),
            full((1, 768)),
            full((1, 768)),
            full((256, 256)),
            full((1, 256)),
        ],
        out_specs=[
            pl.BlockSpec((_ROWS_BLK, 256), lambda i: (i, 0)),
            pl.BlockSpec((_ROWS_BLK, 256), lambda i: (i, 0)),
        ],
        out_shape=[
            jax.ShapeDtypeStruct((_N, 256), jnp.float32),
            jax.ShapeDtypeStruct((_N, 256), jnp.float32),
        ],
    )(s_cat, d0, d1, W2, b2, mix_W, mix_b, Wih, bih, bhh, ro_W, ro_b)


def kernel(x, edge_index, edge_attr, W0, b0, W1, b1, mix_W, mix_b,
           gru_Wih, gru_Whh, gru_bih, gru_bhh, ro_W, ro_b):
    del gru_Whh
    src = edge_index[0]
    dst = edge_index[1]
    ety = edge_attr

    cdst = dst + _N * ety
    csrc = src + _N * ety
    gsrc2 = jnp.concatenate([src, src + _N])
    gdst2 = jnp.concatenate([cdst, cdst + _NR])

    xs = jnp.concatenate([x[:, :64], x[:, 64:]], axis=0)

    acc1, h1 = _sc_pass(xs, gsrc2, cdst)
    t = _scale_stage(acc1, jnp.concatenate([h1, h1])[:, None])
    acc2, h2 = _sc_pass(t, gdst2, csrc)

    s_cat = jnp.concatenate(
        [acc2[0:_N], acc2[_NR:_NR + _N],
         acc2[_N:2 * _N], acc2[_NR + _N:_NR + 2 * _N]],
        axis=1)
    d0 = h2[:_N, None]
    d1 = h2[_N:2 * _N, None]

    W2 = jnp.zeros((256, 512), jnp.float32)
    W2 = W2.at[:128, :256].set(W0).at[128:, 256:].set(W1)
    b2 = jnp.concatenate([b0, b1])[None, :]

    h_next, o = _dense_stage(
        s_cat, d0, d1, W2, b2, mix_W, mix_b[None, :], gru_Wih, gru_bih[None, :],
        gru_bhh[None, :], ro_W, ro_b[None, :])
    return (h_next, o[:, :3])


# speedup vs baseline: 12.8792x; 2.1552x over previous
import functools

import jax
import jax.numpy as jnp
from jax import lax
from jax.experimental import pallas as pl
from jax.experimental.pallas import tpu as pltpu
from jax.experimental.pallas import tpu_sc as plsc

_N = 10000
_E = 320000
_HID = 256
_ROWS_BLK = 2000

_NSC = 2
_NT = 16
_NR = 20480
_RT = _NR // _NT
_EP = _E // _NT
_K = 128
_NCHT = 158
_EPAD = _NT * _NCHT * _K - _E
_SB = 80


def _sc_pass_body(table, pk, out, out_hist, acc, hist, pkA, pkB, rowsA, rowsB,
                  sb, histv, onesv, semA, semB):
    c = lax.axis_index("c")
    s = lax.axis_index("s")
    r0 = s * _RT
    cb = (c * _NT + s) * _NCHT
    z16 = jnp.zeros((16,), jnp.float32)
    one16 = jnp.ones((16,), jnp.float32)
    n_strips = _RT // _SB

    def _zrow(i, carry):
        for j in range(4):
            sb[i, pl.ds(j * 16, 16)] = z16
        return carry
    lax.fori_loop(0, _SB, _zrow, 0)

    def _zhist(i, carry):
        histv[pl.ds(i * 16, 16)] = z16
        return carry
    lax.fori_loop(0, _RT // 16, _zhist, 0)
    for j in range(_K // 16):
        onesv[pl.ds(j * 16, 16)] = one16

    def _zstrip(st, carry):
        pltpu.sync_copy(sb, acc.at[pl.ds(r0 + st * _SB, _SB)])
        return carry
    lax.fori_loop(0, n_strips, _zstrip, 0)
    pltpu.sync_copy(histv, hist.at[pl.ds(r0, _RT)])
    plsc.subcore_barrier()

    pltpu.sync_copy(pk.at[cb], pkA)
    pltpu.async_copy(table.at[pkA.at[0]], rowsA, semA)
    pltpu.sync_copy(pk.at[cb + 1], pkB)
    pltpu.async_copy(table.at[pkB.at[0]], rowsB, semB)

    def _pair(p, carry):
        a = 2 * p
        pltpu.make_async_copy(table.at[pl.ds(0, _K)], rowsA, semA).wait()
        pltpu.sync_copy(rowsA, acc.at[pkA.at[1]], add=True)

        @pl.when(c == 0)
        def _():
            pltpu.sync_copy(onesv, hist.at[pkA.at[1]], add=True)

        @pl.when(a + 2 < _NCHT)
        def _():
            pltpu.sync_copy(pk.at[cb + a + 2], pkA)
            pltpu.async_copy(table.at[pkA.at[0]], rowsA, semA)

        pltpu.make_async_copy(table.at[pl.ds(0, _K)], rowsB, semB).wait()
        pltpu.sync_copy(rowsB, acc.at[pkB.at[1]], add=True)

        @pl.when(c == 1)
        def _():
            pltpu.sync_copy(onesv, hist.at[pkB.at[1]], add=True)

        @pl.when(a + 3 < _NCHT)
        def _():
            pltpu.sync_copy(pk.at[cb + a + 3], pkB)
            pltpu.async_copy(table.at[pkB.at[0]], rowsB, semB)
        return carry
    lax.fori_loop(0, _NCHT // 2, _pair, 0)

    plsc.subcore_barrier()

    def _fstrip(st, carry):
        pltpu.sync_copy(acc.at[pl.ds(r0 + st * _SB, _SB)], sb)
        pltpu.sync_copy(sb, out.at[pl.ds(c * _NR + r0 + st * _SB, _SB)])
        return carry
    lax.fori_loop(0, n_strips, _fstrip, 0)

    pltpu.sync_copy(hist.at[pl.ds(r0, _RT)], histv)
    pltpu.sync_copy(histv, out_hist.at[pl.ds(c * _NR + r0, _RT)])


def _sc_pass(table, pk):
    mesh = plsc.VectorSubcoreMesh(core_axis_name="c", subcore_axis_name="s")
    f = pl.kernel(
        _sc_pass_body,
        mesh=mesh,
        out_type=[
            jax.ShapeDtypeStruct((_NSC * _NR, 64), jnp.float32),
            jax.ShapeDtypeStruct((_NSC * _NR,), jnp.float32),
        ],
        scratch_types=[
            pltpu.VMEM_SHARED((_NR, 64), jnp.float32),
            pltpu.VMEM_SHARED((_NR,), jnp.float32),
            pltpu.VMEM((2, _K), jnp.int32),
            pltpu.VMEM((2, _K), jnp.int32),
            pltpu.VMEM((_K, 64), jnp.float32),
            pltpu.VMEM((_K, 64), jnp.float32),
            pltpu.VMEM((_SB, 64), jnp.float32),
            pltpu.VMEM((_RT,), jnp.float32),
            pltpu.VMEM((_K,), jnp.float32),
            pltpu.SemaphoreType.DMA,
            pltpu.SemaphoreType.DMA,
        ],
        compiler_params=pltpu.CompilerParams(use_tc_tiling_on_sc=False),
    )
    return f(table, pk)


def _scale_body(a_ref, h_ref, o_ref):
    h = h_ref[...]
    o_ref[...] = a_ref[...] * jnp.where(h > 0.0, 1.0 / h, 0.0)


def _scale_stage(acc, histcat):
    blk = 4096
    grid = (_NSC * _NR // blk,)
    return pl.pallas_call(
        _scale_body,
        grid=grid,
        in_specs=[
            pl.BlockSpec((blk, 64), lambda i: (i, 0)),
            pl.BlockSpec((blk, 1), lambda i: (i, 0)),
        ],
        out_specs=pl.BlockSpec((blk, 64), lambda i: (i, 0)),
        out_shape=jax.ShapeDtypeStruct((_NSC * _NR, 64), jnp.float32),
    )(acc, histcat)


def _dense_body(s_ref, d0_ref, d1_ref, W2_ref, b2_ref, mixW_ref, mixb_ref,
                Wih_ref, bih_ref, bhh_ref, roW_ref, rob_ref, h_ref, o_ref):
    d0 = d0_ref[...]
    d1 = d1_ref[...]
    inv0 = jnp.where(d0 > 0.0, 1.0 / d0, 0.0)
    inv1 = jnp.where(d1 > 0.0, 1.0 / d1, 0.0)
    s0 = s_ref[...]
    s = jnp.concatenate([s0[:, :128] * inv0, s0[:, 128:] * inv1], axis=1)
    u = jnp.dot(s, W2_ref[...], preferred_element_type=jnp.float32) + b2_ref[...]
    h = jnp.maximum(
        jnp.dot(u, mixW_ref[...], preferred_element_type=jnp.float32) + mixb_ref[...],
        0.0)
    gi = jnp.dot(h, Wih_ref[...], preferred_element_type=jnp.float32) + bih_ref[...]
    bhh = bhh_ref[...]
    r = jax.nn.sigmoid(gi[:, 0:_HID] + bhh[:, 0:_HID])
    z = jax.nn.sigmoid(gi[:, _HID:2 * _HID] + bhh[:, _HID:2 * _HID])
    n = jnp.tanh(gi[:, 2 * _HID:] + r * bhh[:, 2 * _HID:])
    hn = (1.0 - z) * n
    h_ref[...] = hn
    o_ref[...] = jnp.dot(hn, roW_ref[...], preferred_element_type=jnp.float32) + rob_ref[...]


def _dense_stage(s_cat, d0, d1, W2, b2, mix_W, mix_b, Wih, bih, bhh, ro_W, ro_b):
    grid = (_N // _ROWS_BLK,)
    full = lambda shape: pl.BlockSpec(shape, lambda i: (0, 0))
    return pl.pallas_call(
        _dense_body,
        grid=grid,
        in_specs=[
            pl.BlockSpec((_ROWS_BLK, 256), lambda i: (i, 0)),
            pl.BlockSpec((_ROWS_BLK, 1), lambda i: (i, 0)),
            pl.BlockSpec((_ROWS_BLK, 1), lambda i: (i, 0)),
            full((256, 512)),
            full((1, 512)),
            full((512, 256)),
            full((1, 256)),
            full((256, 768)),
            full((1, 768)),
            full((1, 768)),
            full((256, 256)),
            full((1, 256)),
        ],
        out_specs=[
            pl.BlockSpec((_ROWS_BLK, 256), lambda i: (i, 0)),
            pl.BlockSpec((_ROWS_BLK, 256), lambda i: (i, 0)),
        ],
        out_shape=[
            jax.ShapeDtypeStruct((_N, 256), jnp.float32),
            jax.ShapeDtypeStruct((_N, 256), jnp.float32),
        ],
    )(s_cat, d0, d1, W2, b2, mix_W, mix_b, Wih, bih, bhh, ro_W, ro_b)


def kernel(x, edge_index, edge_attr, W0, b0, W1, b1, mix_W, mix_b,
           gru_Wih, gru_Whh, gru_bih, gru_bhh, ro_W, ro_b):
    del gru_Whh
    src = edge_index[0]
    dst = edge_index[1]
    ety = edge_attr

    cdst = dst + _N * ety
    csrc = src + _N * ety

    def _pack(g, sidx, goff1):
        gp = jnp.concatenate(
            [g, jnp.arange(_EPAD, dtype=jnp.int32) % _N])
        sp = jnp.concatenate(
            [sidx, 2 * _N + jnp.arange(_EPAD, dtype=jnp.int32) % (_NR - 2 * _N)])
        g2 = jnp.stack([gp, gp + goff1]).reshape(2, _NT * _NCHT, 1, _K)
        s2 = jnp.broadcast_to(
            sp.reshape(1, _NT * _NCHT, 1, _K), (2, _NT * _NCHT, 1, _K))
        return jnp.concatenate([g2, s2], axis=2).reshape(-1, 2, _K)

    pk1 = _pack(src, cdst, _N)
    pk2 = _pack(cdst, csrc, _NR)

    xs = jnp.concatenate([x[:, :64], x[:, 64:]], axis=0)

    acc1, h1p = _sc_pass(xs, pk1)
    h1 = h1p[:_NR] + h1p[_NR:]
    t = _scale_stage(acc1, jnp.concatenate([h1, h1])[:, None])
    acc2, h2p = _sc_pass(t, pk2)
    h2 = h2p[:_NR] + h2p[_NR:]

    s_cat = jnp.concatenate(
        [acc2[0:_N], acc2[_NR:_NR + _N],
         acc2[_N:2 * _N], acc2[_NR + _N:_NR + 2 * _N]],
        axis=1)
    d0 = h2[:_N, None]
    d1 = h2[_N:2 * _N, None]

    W2 = jnp.zeros((256, 512), jnp.float32)
    W2 = W2.at[:128, :256].set(W0).at[128:, 256:].set(W1)
    b2 = jnp.concatenate([b0, b1])[None, :]

    h_next, o = _dense_stage(
        s_cat, d0, d1, W2, b2, mix_W, mix_b[None, :], gru_Wih, gru_bih[None, :],
        gru_bhh[None, :], ro_W, ro_b[None, :])
    return (h_next, o[:, :3])


# speedup vs baseline: 13.6111x; 1.0568x over previous
import functools

import jax
import jax.numpy as jnp
from jax import lax
from jax.experimental import pallas as pl
from jax.experimental.pallas import tpu as pltpu
from jax.experimental.pallas import tpu_sc as plsc

_N = 10000
_E = 320000
_HID = 256
_ROWS_BLK = 2000

_NSC = 2
_NT = 16
_NR = 20480
_RT = _NR // _NT
_EP = _E // _NT
_K = 128
_NCHT = 158
_EPAD = _NT * _NCHT * _K - _E
_SB = 80


def _sc_fused_body(xs, pk1, pk2, s_out, t_out, acc, hist, pkA, pkB, rowsA,
                   rowsB, sb, histv, onesv, semA, semB):
    c = lax.axis_index("c")
    s = lax.axis_index("s")
    r0 = s * _RT
    cb = (c * _NT + s) * _NCHT
    z16 = jnp.zeros((16,), jnp.float32)
    one16 = jnp.ones((16,), jnp.float32)
    n_strips = _RT // _SB
    lane_splats = [jnp.full((16, 1), r, jnp.int32) for r in range(16)]
    _gd = lax.GatherDimensionNumbers(
        offset_dims=(), collapsed_slice_dims=(0,), start_index_map=(0,))

    def _zero_acc_hist():
        def _zstrip(st, carry):
            pltpu.sync_copy(rowsA.at[pl.ds(0, _SB)],
                            acc.at[pl.ds(r0 + st * _SB, _SB)])
            return carry
        lax.fori_loop(0, n_strips, _zstrip, 0)
        pltpu.sync_copy(histv, hist.at[pl.ds(r0, _RT)])

    def _edge_loop(table, pk):
        pltpu.sync_copy(pk.at[cb], pkA)
        pltpu.async_copy(table.at[pkA.at[0]], rowsA, semA)
        pltpu.sync_copy(pk.at[cb + 1], pkB)
        pltpu.async_copy(table.at[pkB.at[0]], rowsB, semB)

        def _pair(p, carry):
            a = 2 * p
            pltpu.make_async_copy(table.at[pl.ds(0, _K)], rowsA, semA).wait()
            pltpu.sync_copy(rowsA, acc.at[pkA.at[1]], add=True)
            pltpu.sync_copy(onesv, hist.at[pkA.at[1]], add=True)

            @pl.when(a + 2 < _NCHT)
            def _():
                pltpu.sync_copy(pk.at[cb + a + 2], pkA)
                pltpu.async_copy(table.at[pkA.at[0]], rowsA, semA)

            pltpu.make_async_copy(table.at[pl.ds(0, _K)], rowsB, semB).wait()
            pltpu.sync_copy(rowsB, acc.at[pkB.at[1]], add=True)
            pltpu.sync_copy(onesv, hist.at[pkB.at[1]], add=True)

            @pl.when(a + 3 < _NCHT)
            def _():
                pltpu.sync_copy(pk.at[cb + a + 3], pkB)
                pltpu.async_copy(table.at[pkB.at[0]], rowsB, semB)
            return carry
        lax.fori_loop(0, _NCHT // 2, _pair, 0)

    def _finalize(dst, rezero):
        if rezero:
            def _rz(i, carry):
                for j in range(4):
                    rowsA[i, pl.ds(j * 16, 16)] = z16
                return carry
            lax.fori_loop(0, _SB, _rz, 0)
        pltpu.sync_copy(hist.at[pl.ds(r0, _RT)], histv)

        def _inv(g, carry):
            hv = histv[pl.ds(g * 16, 16)]
            histv[pl.ds(g * 16, 16)] = jnp.where(hv > 0.0, 1.0 / hv, 0.0)
            return carry
        lax.fori_loop(0, _RT // 16, _inv, 0)

        def _fstrip(st, carry):
            pltpu.sync_copy(acc.at[pl.ds(r0 + st * _SB, _SB)], sb)

            def _grp(g, carry2):
                inv16 = histv[pl.ds(st * _SB + g * 16, 16)]
                for r in range(16):
                    splat = lax.gather(
                        inv16, lane_splats[r], _gd, slice_sizes=(1,),
                        mode=lax.GatherScatterMode.PROMISE_IN_BOUNDS)
                    row = g * 16 + r
                    for j in range(4):
                        sb[row, pl.ds(j * 16, 16)] = sb[row, pl.ds(j * 16, 16)] * splat
                return carry2
            lax.fori_loop(0, _SB // 16, _grp, 0)
            pltpu.sync_copy(sb, dst.at[pl.ds(c * _NR + r0 + st * _SB, _SB)])
            if rezero:
                pltpu.sync_copy(rowsA.at[pl.ds(0, _SB)],
                                acc.at[pl.ds(r0 + st * _SB, _SB)])
            return carry
        lax.fori_loop(0, n_strips, _fstrip, 0)
        if rezero:
            def _zh(g, carry):
                histv[pl.ds(g * 16, 16)] = z16
                return carry
            lax.fori_loop(0, _RT // 16, _zh, 0)
            pltpu.sync_copy(histv, hist.at[pl.ds(r0, _RT)])

    def _zrow(i, carry):
        for j in range(4):
            rowsA[i, pl.ds(j * 16, 16)] = z16
        return carry
    lax.fori_loop(0, _K, _zrow, 0)

    def _zhist(i, carry):
        histv[pl.ds(i * 16, 16)] = z16
        return carry
    lax.fori_loop(0, _RT // 16, _zhist, 0)
    for j in range(_K // 16):
        onesv[pl.ds(j * 16, 16)] = one16
    _zero_acc_hist()
    plsc.subcore_barrier()

    _edge_loop(xs, pk1)
    plsc.subcore_barrier()
    _finalize(t_out, rezero=True)
    plsc.subcore_barrier()

    _edge_loop(t_out, pk2)
    plsc.subcore_barrier()
    _finalize(s_out, rezero=False)


def _sc_fused(xs, pk1, pk2):
    mesh = plsc.VectorSubcoreMesh(core_axis_name="c", subcore_axis_name="s")
    f = pl.kernel(
        _sc_fused_body,
        mesh=mesh,
        out_type=[
            jax.ShapeDtypeStruct((_NSC * _NR, 64), jnp.float32),
            jax.ShapeDtypeStruct((_NSC * _NR, 64), jnp.float32),
        ],
        scratch_types=[
            pltpu.VMEM_SHARED((_NR, 64), jnp.float32),
            pltpu.VMEM_SHARED((_NR,), jnp.float32),
            pltpu.VMEM((2, _K), jnp.int32),
            pltpu.VMEM((2, _K), jnp.int32),
            pltpu.VMEM((_K, 64), jnp.float32),
            pltpu.VMEM((_K, 64), jnp.float32),
            pltpu.VMEM((_SB, 64), jnp.float32),
            pltpu.VMEM((_RT,), jnp.float32),
            pltpu.VMEM((_K,), jnp.float32),
            pltpu.SemaphoreType.DMA,
            pltpu.SemaphoreType.DMA,
        ],
        compiler_params=pltpu.CompilerParams(use_tc_tiling_on_sc=False),
    )
    return f(xs, pk1, pk2)


def _dense_body(s_ref, W2_ref, b2_ref, mixW_ref, mixb_ref,
                Wih_ref, bih_ref, bhh_ref, roW_ref, rob_ref, h_ref, o_ref):
    s = s_ref[...]
    u = jnp.dot(s, W2_ref[...], preferred_element_type=jnp.float32) + b2_ref[...]
    h = jnp.maximum(
        jnp.dot(u, mixW_ref[...], preferred_element_type=jnp.float32) + mixb_ref[...],
        0.0)
    gi = jnp.dot(h, Wih_ref[...], preferred_element_type=jnp.float32) + bih_ref[...]
    bhh = bhh_ref[...]
    r = jax.nn.sigmoid(gi[:, 0:_HID] + bhh[:, 0:_HID])
    z = jax.nn.sigmoid(gi[:, _HID:2 * _HID] + bhh[:, _HID:2 * _HID])
    n = jnp.tanh(gi[:, 2 * _HID:] + r * bhh[:, 2 * _HID:])
    hn = (1.0 - z) * n
    h_ref[...] = hn
    o_ref[...] = jnp.dot(hn, roW_ref[...], preferred_element_type=jnp.float32) + rob_ref[...]


def _dense_stage(s_cat, W2, b2, mix_W, mix_b, Wih, bih, bhh, ro_W, ro_b):
    grid = (_N // _ROWS_BLK,)
    full = lambda shape: pl.BlockSpec(shape, lambda i: (0, 0))
    return pl.pallas_call(
        _dense_body,
        grid=grid,
        in_specs=[
            pl.BlockSpec((_ROWS_BLK, 256), lambda i: (i, 0)),
            full((256, 512)),
            full((1, 512)),
            full((512, 256)),
            full((1, 256)),
            full((256, 768)),
            full((1, 768)),
            full((1, 768)),
            full((256, 256)),
            full((1, 256)),
        ],
        out_specs=[
            pl.BlockSpec((_ROWS_BLK, 256), lambda i: (i, 0)),
            pl.BlockSpec((_ROWS_BLK, 256), lambda i: (i, 0)),
        ],
        out_shape=[
            jax.ShapeDtypeStruct((_N, 256), jnp.float32),
            jax.ShapeDtypeStruct((_N, 256), jnp.float32),
        ],
    )(s_cat, W2, b2, mix_W, mix_b, Wih, bih, bhh, ro_W, ro_b)


def kernel(x, edge_index, edge_attr, W0, b0, W1, b1, mix_W, mix_b,
           gru_Wih, gru_Whh, gru_bih, gru_bhh, ro_W, ro_b):
    del gru_Whh
    src = edge_index[0]
    dst = edge_index[1]
    ety = edge_attr

    cdst = dst + _N * ety
    csrc = src + _N * ety

    def _pack(g, sidx, goff1):
        gp = jnp.concatenate(
            [g, jnp.arange(_EPAD, dtype=jnp.int32) % _N])
        sp = jnp.concatenate(
            [sidx, 2 * _N + jnp.arange(_EPAD, dtype=jnp.int32) % (_NR - 2 * _N)])
        g2 = jnp.stack([gp, gp + goff1]).reshape(2, _NT * _NCHT, 1, _K)
        s2 = jnp.broadcast_to(
            sp.reshape(1, _NT * _NCHT, 1, _K), (2, _NT * _NCHT, 1, _K))
        return jnp.concatenate([g2, s2], axis=2).reshape(-1, 2, _K)

    pk1 = _pack(src, cdst, _N)
    pk2 = _pack(cdst, csrc, _NR)

    xs = jnp.concatenate([x[:, :64], x[:, 64:]], axis=0)

    s, _t = _sc_fused(xs, pk1, pk2)

    s_cat = jnp.concatenate(
        [s[0:_N], s[_NR:_NR + _N], s[_N:2 * _N], s[_NR + _N:_NR + 2 * _N]],
        axis=1)

    W2 = jnp.zeros((256, 512), jnp.float32)
    W2 = W2.at[:128, :256].set(W0).at[128:, 256:].set(W1)
    b2 = jnp.concatenate([b0, b1])[None, :]

    h_next, o = _dense_stage(
        s_cat, W2, b2, mix_W, mix_b[None, :], gru_Wih, gru_bih[None, :],
        gru_bhh[None, :], ro_W, ro_b[None, :])
    return (h_next, o[:, :3])
